# baseline (speedup 1.0000x reference)
import functools

import jax
import jax.numpy as jnp
from jax import lax
from jax.experimental import pallas as pl
from jax.experimental.pallas import tpu as pltpu
from jax.experimental.pallas import tpu_sc as plsc

_B = 64
_HW = 196
_C = 768

_SC_WORKERS = 8
_SC_BPW = _B // _SC_WORKERS


def _sc_gather_rows(labels, gamma, beta):
    mesh = plsc.VectorSubcoreMesh(core_axis_name="c", subcore_axis_name="s")

    @functools.partial(
        pl.kernel,
        mesh=mesh,
        out_type=[
            jax.ShapeDtypeStruct((_B, _C), jnp.float32),
            jax.ShapeDtypeStruct((_B, _C), jnp.float32),
        ],
        scratch_types=[
            pltpu.VMEM((_SC_BPW,), jnp.int32),
            pltpu.VMEM((_SC_BPW, _C), jnp.float32),
            pltpu.VMEM((_SC_BPW, _C), jnp.float32),
            pltpu.SemaphoreType.DMA,
            pltpu.SemaphoreType.DMA,
        ],
    )
    def gather_kernel(labels_hbm, gamma_hbm, beta_hbm, g_out, b_out,
                      idx_v, g_rows, b_rows, sem_g, sem_b):
        wid = lax.axis_index("s") * 2 + lax.axis_index("c")

        @pl.when(wid < _SC_WORKERS)
        def _():
            base = wid * _SC_BPW
            pltpu.sync_copy(labels_hbm.at[pl.ds(base, _SC_BPW)], idx_v)
            cg = pltpu.async_copy(gamma_hbm.at[idx_v], g_rows, sem_g)
            cb = pltpu.async_copy(beta_hbm.at[idx_v], b_rows, sem_b)
            cg.wait()
            pltpu.sync_copy(g_rows, g_out.at[pl.ds(base, _SC_BPW)])
            cb.wait()
            pltpu.sync_copy(b_rows, b_out.at[pl.ds(base, _SC_BPW)])

    return gather_kernel(labels, gamma, beta)


def _tc_scale_shift(x3, g_rows, b_rows, block_b):
    def body(x_ref, g_ref, b_ref, o_ref):
        o_ref[...] = (
            x_ref[...] * g_ref[...][:, None, :] + b_ref[...][:, None, :]
        )

    return pl.pallas_call(
        body,
        grid=(_B // block_b,),
        in_specs=[
            pl.BlockSpec((block_b, _HW, _C), lambda i: (i, 0, 0)),
            pl.BlockSpec((block_b, _C), lambda i: (i, 0)),
            pl.BlockSpec((block_b, _C), lambda i: (i, 0)),
        ],
        out_specs=pl.BlockSpec((block_b, _HW, _C), lambda i: (i, 0, 0)),
        out_shape=jax.ShapeDtypeStruct((_B, _HW, _C), jnp.float32),
    )(x3, g_rows, b_rows)


@jax.jit
def kernel(x, class_labels, gamma, beta):
    labels = class_labels.reshape(_B)
    g_rows, b_rows = _sc_gather_rows(labels, gamma, beta)
    x3 = x.reshape(_B, _HW, _C)
    out = _tc_scale_shift(x3, g_rows, b_rows, block_b=8)
    return out.reshape(x.shape)
